# initial kernel scaffold (unmeasured)
import jax
import jax.numpy as jnp
from jax import lax
from jax.experimental import pallas as pl
from jax.experimental.pallas import tpu as pltpu


def kernel(Q, K, V):
    b, sq, h, d = Q.shape
    skv = K.shape[1]
    hd = h * d
    scale = d ** -0.5

    Qf = Q.reshape(b, hd)
    Kf = K.reshape(b, skv, hd)
    Vf = V.reshape(b, skv, hd)

    def body(q_ref, k_ref, v_ref, o_ref,
             acc_ref, l_ref, racc_ref, rl_ref, send_sems, recv_sems):
        bi = pl.program_id(0)
        nb = pl.num_programs(0)
        my_x = lax.axis_index("x")
        my_y = lax.axis_index("y")
        nbr = (my_x, 1 - my_y)

        @pl.when(bi == 0)
        def _():
            barrier_sem = pltpu.get_barrier_semaphore()
            pl.semaphore_signal(
                barrier_sem, inc=1,
                device_id=nbr, device_id_type=pl.DeviceIdType.MESH,
            )
            pl.semaphore_wait(barrier_sem, 1)

        rows = lax.broadcasted_iota(jnp.int32, (h, hd), 0)
        cols = lax.broadcasted_iota(jnp.int32, (h, hd), 1)
        mask = (cols // d) == rows
        maskf = mask.astype(jnp.float32)

        q = q_ref[...] * scale
        qd = jnp.where(mask, q, 0.0)
        k2 = k_ref[0]
        v2 = v_ref[0]

        s = lax.dot_general(qd, k2, (((1,), (1,)), ((), ())),
                            preferred_element_type=jnp.float32)
        p = jnp.exp(s)
        lrow = lax.dot_general(jnp.ones((1, skv), jnp.float32), p,
                               (((1,), (1,)), ((), ())),
                               preferred_element_type=jnp.float32)
        r = lax.dot_general(p, v2, (((1,), (0,)), ((), ())),
                            preferred_element_type=jnp.float32)
        accrow = jnp.sum(r * maskf, axis=0, keepdims=True)

        acc_ref[pl.ds(bi, 1), :] = accrow
        l_ref[pl.ds(bi, 1), :] = lrow

        @pl.when(bi == nb - 1)
        def _():
            rdma_acc = pltpu.make_async_remote_copy(
                src_ref=acc_ref, dst_ref=racc_ref,
                send_sem=send_sems.at[0], recv_sem=recv_sems.at[0],
                device_id=nbr, device_id_type=pl.DeviceIdType.MESH,
            )
            rdma_l = pltpu.make_async_remote_copy(
                src_ref=l_ref, dst_ref=rl_ref,
                send_sem=send_sems.at[1], recv_sem=recv_sems.at[1],
                device_id=nbr, device_id_type=pl.DeviceIdType.MESH,
            )
            rdma_acc.start()
            rdma_l.start()
            rdma_acc.wait()
            rdma_l.wait()

            lsum = l_ref[...] + rl_ref[...]
            lfull = lax.dot_general(lsum, maskf, (((1,), (0,)), ((), ())),
                                    preferred_element_type=jnp.float32)
            o_ref[...] = (acc_ref[...] + racc_ref[...]) / lfull

    out = pl.pallas_call(
        body,
        grid=(b,),
        out_shape=jax.ShapeDtypeStruct((b, hd), jnp.float32),
        in_specs=[
            pl.BlockSpec((1, hd), lambda i: (i, 0)),
            pl.BlockSpec((1, skv, hd), lambda i: (i, 0, 0)),
            pl.BlockSpec((1, skv, hd), lambda i: (i, 0, 0)),
        ],
        out_specs=pl.BlockSpec((b, hd), lambda i: (0, 0)),
        scratch_shapes=[
            pltpu.VMEM((b, hd), jnp.float32),
            pltpu.VMEM((b, h), jnp.float32),
            pltpu.VMEM((b, hd), jnp.float32),
            pltpu.VMEM((b, h), jnp.float32),
            pltpu.SemaphoreType.DMA((2,)),
            pltpu.SemaphoreType.DMA((2,)),
        ],
        compiler_params=pltpu.CompilerParams(collective_id=0),
    )(Qf, Kf, Vf)
    return out.reshape(b, sq, h, d)


# baseline (device time: 175619 ns/iter reference)
import jax
import jax.numpy as jnp
from jax import lax
from jax.experimental import pallas as pl
from jax.experimental.pallas import tpu as pltpu


def kernel(Q, K, V):
    b, sq, h, d = Q.shape
    skv = K.shape[1]
    hd = h * d
    scale = d ** -0.5

    Qf = Q.reshape(b, hd)
    Kf = K.reshape(b, skv, hd)
    Vf = V.reshape(b, skv, hd)

    def body(q_ref, k_ref, v_ref, o_ref,
             acc_ref, l_ref, racc_ref, rl_ref, send_sems, recv_sems):
        bi = pl.program_id(0)
        nb = pl.num_programs(0)
        my_x = lax.axis_index("x")
        my_y = lax.axis_index("y")
        nbr = (my_x, 1 - my_y)

        @pl.when(bi == 0)
        def _():
            barrier_sem = pltpu.get_barrier_semaphore()
            pl.semaphore_signal(
                barrier_sem, inc=1,
                device_id=nbr, device_id_type=pl.DeviceIdType.MESH,
            )
            pl.semaphore_wait(barrier_sem, 1)

        rows = lax.broadcasted_iota(jnp.int32, (h, hd), 0)
        cols = lax.broadcasted_iota(jnp.int32, (h, hd), 1)
        mask = (cols // d) == rows
        maskf = mask.astype(jnp.float32)

        q = q_ref[pl.ds(bi, 1), :] * scale
        qd = jnp.where(mask, q, 0.0)
        k2 = k_ref[0]
        v2 = v_ref[0]

        s = lax.dot_general(qd, k2, (((1,), (1,)), ((), ())),
                            preferred_element_type=jnp.float32)
        p = jnp.exp(s)
        lrow = lax.dot_general(jnp.ones((1, skv), jnp.float32), p,
                               (((1,), (1,)), ((), ())),
                               preferred_element_type=jnp.float32)
        r = lax.dot_general(p, v2, (((1,), (0,)), ((), ())),
                            preferred_element_type=jnp.float32)
        accrow = jnp.sum(r * maskf, axis=0, keepdims=True)

        acc_ref[pl.ds(bi, 1), :] = accrow
        l_ref[pl.ds(bi, 1), :] = lrow

        @pl.when(bi == nb - 1)
        def _():
            rdma_acc = pltpu.make_async_remote_copy(
                src_ref=acc_ref, dst_ref=racc_ref,
                send_sem=send_sems.at[0], recv_sem=recv_sems.at[0],
                device_id=nbr, device_id_type=pl.DeviceIdType.MESH,
            )
            rdma_l = pltpu.make_async_remote_copy(
                src_ref=l_ref, dst_ref=rl_ref,
                send_sem=send_sems.at[1], recv_sem=recv_sems.at[1],
                device_id=nbr, device_id_type=pl.DeviceIdType.MESH,
            )
            rdma_acc.start()
            rdma_l.start()
            rdma_acc.wait()
            rdma_l.wait()

            lsum = l_ref[...] + rl_ref[...]
            lfull = lax.dot_general(lsum, maskf, (((1,), (0,)), ((), ())),
                                    preferred_element_type=jnp.float32)
            o_ref[...] = (acc_ref[...] + racc_ref[...]) / lfull

    out = pl.pallas_call(
        body,
        grid=(b,),
        out_shape=jax.ShapeDtypeStruct((b, hd), jnp.float32),
        in_specs=[
            pl.BlockSpec((b, hd), lambda i: (0, 0)),
            pl.BlockSpec((1, skv, hd), lambda i: (i, 0, 0)),
            pl.BlockSpec((1, skv, hd), lambda i: (i, 0, 0)),
        ],
        out_specs=pl.BlockSpec((b, hd), lambda i: (0, 0)),
        scratch_shapes=[
            pltpu.VMEM((b, hd), jnp.float32),
            pltpu.VMEM((b, h), jnp.float32),
            pltpu.VMEM((b, hd), jnp.float32),
            pltpu.VMEM((b, h), jnp.float32),
            pltpu.SemaphoreType.DMA((2,)),
            pltpu.SemaphoreType.DMA((2,)),
        ],
        compiler_params=pltpu.CompilerParams(collective_id=0),
    )(Qf, Kf, Vf)
    return out.reshape(b, sq, h, d)
